# baseline (speedup 1.0000x reference)
import jax
import jax.numpy as jnp
from jax import lax
from jax.experimental import pallas as pl
from jax.experimental.pallas import tpu as pltpu
from jax.experimental.pallas import tpu_sc as plsc

B, L, D, N = 4, 2048, 4096, 16
NEW_L = L + N
NC, NS = 2, 16
NW = NC * NS
SUBS_PER_B = NW // B
LANES = 16

H = 2
DW = D // H
HL = L * H
HNEW = NEW_L * H
SHIFT = N * H
HR_PER_W = HL // SUBS_PER_B
CHUNK = 8
NCHUNK = HR_PER_W // CHUNK
NSLOT = 4
UNROLL = NSLOT


def _sc_body(emb_hbm, mem_hbm, pos_hbm, out_hbm, buf, pos_v, gsems, ssems):
    c = lax.axis_index("c")
    s = lax.axis_index("s")
    wid = c * NS + s
    b = wid // SUBS_PER_B
    sub = wid % SUBS_PER_B

    base_local = sub * HR_PER_W
    src_base = b * HL + base_local
    out_base = b * HNEW
    iota = lax.iota(jnp.int32, LANES)

    slots = [buf.at[pl.ds(k * CHUNK, CHUNK)] for k in range(NSLOT)]

    def start_gather(i, k):
        r = pl.multiple_of(src_base + i * CHUNK, CHUNK)
        pltpu.async_copy(emb_hbm.at[pl.ds(r, CHUNK)], slots[k], gsems.at[k])

    def wait_gather(k):
        pltpu.make_async_copy(emb_hbm.at[pl.ds(0, CHUNK)], slots[k],
                              gsems.at[k]).wait()

    start_gather(0, 0)
    start_gather(1, 1)
    pltpu.sync_copy(pos_hbm, pos_v)
    pos_s = pos_v[pl.ds(b, LANES)][0]
    pos_h = pos_s * H
    pos_hvec = jnp.full((LANES,), pos_h, jnp.int32)

    def scatter_each(i, k, fn):
        row0 = base_local + i * CHUNK

        @pl.when(row0 < pos_h)
        def _():
            fn(slots[k], pl.multiple_of(out_base + row0, CHUNK), ssems.at[k])

        @pl.when(row0 + CHUNK > pos_h)
        def _():
            fn(slots[k], pl.multiple_of(out_base + row0 + SHIFT, CHUNK),
               ssems.at[k])

    def start_scatter(i, k):
        scatter_each(
            i, k, lambda sl, dst0, sem:
            pltpu.async_copy(sl, out_hbm.at[pl.ds(dst0, CHUNK)], sem))

    def wait_scatter(i, k):
        scatter_each(
            i, k, lambda sl, dst0, sem:
            pltpu.make_async_copy(sl, out_hbm.at[pl.ds(dst0, CHUNK)],
                                  sem).wait())

    def body(t, _):
        i0 = t * UNROLL
        for k in range(UNROLL):
            i = i0 + k
            wait_gather(k)
            start_scatter(i, k)

            @pl.when(i >= 2)
            def _():
                wait_scatter(i - 2, (k + 2) % NSLOT)

            @pl.when(i + 2 < NCHUNK)
            def _():
                start_gather(i + 2, (k + 2) % NSLOT)
        return 0
    lax.fori_loop(0, NCHUNK // UNROLL, body, 0)
    wait_scatter(NCHUNK - 2, (NCHUNK - 2) % NSLOT)
    wait_scatter(NCHUNK - 1, (NCHUNK - 1) % NSLOT)

    @pl.when(sub == pos_h // HR_PER_W)
    def _():
        mrow0 = pl.multiple_of(b * N * H, CHUNK)
        stage = buf.at[pl.ds(0, 2 * LANES)]
        pltpu.sync_copy(mem_hbm.at[pl.ds(mrow0, 2 * LANES)], stage)
        for k in range(2):
            dstm = out_base + pos_hvec + k * LANES + iota
            pltpu.sync_copy(stage.at[pl.ds(k * LANES, LANES)],
                            out_hbm.at[dstm])


def _mask_body(am_ref, pos_ref, out_ref):
    j = lax.broadcasted_iota(jnp.int32, (B, NEW_L), 1)
    pos = pos_ref[...].reshape(B, 1)
    am = am_ref[...]
    zpad = jnp.zeros((B, N), jnp.float32)
    am_lo = jnp.concatenate([am, zpad], axis=1)
    am_hi = jnp.concatenate([zpad, am], axis=1)
    out_ref[...] = jnp.where(
        j < pos, am_lo, jnp.where(j >= pos + N, am_hi,
                                  jnp.ones((B, NEW_L), jnp.float32)))


@jax.jit
def kernel(inputs_embeds, memory, attention_mask, injection_positions):
    emb_flat = inputs_embeds.reshape(B * HL, DW)
    mem_flat = memory.reshape(B * N * H, DW)
    am = attention_mask.astype(jnp.float32)
    pos32 = injection_positions.astype(jnp.int32)
    pos_pad = jnp.zeros((2 * LANES,), jnp.int32).at[:B].set(pos32)

    mesh = plsc.VectorSubcoreMesh(core_axis_name="c", subcore_axis_name="s",
                                  num_cores=NC, num_subcores=NS)
    run = pl.kernel(
        _sc_body,
        out_type=jax.ShapeDtypeStruct((B * HNEW, DW), jnp.float32),
        mesh=mesh,
        scratch_types=[
            pltpu.VMEM((NSLOT * CHUNK, DW), jnp.float32),
            pltpu.VMEM((2 * LANES,), jnp.int32),
            pltpu.SemaphoreType.DMA((NSLOT,)),
            pltpu.SemaphoreType.DMA((NSLOT,)),
        ],
    )
    out_flat = run(emb_flat, mem_flat, pos_pad)

    new_mask = pl.pallas_call(
        _mask_body,
        out_shape=jax.ShapeDtypeStruct((B, NEW_L), jnp.float32),
    )(am, pos32)

    return out_flat.reshape(B, NEW_L, D), new_mask


# speedup vs baseline: 3.2114x; 3.2114x over previous
import jax
import jax.numpy as jnp
from jax import lax
from jax.experimental import pallas as pl
from jax.experimental.pallas import tpu as pltpu
from jax.experimental.pallas import tpu_sc as plsc

B, L, D, N = 4, 2048, 4096, 16
NEW_L = L + N
NC, NS = 2, 16
NW = NC * NS
SUBS_PER_B = NW // B
ROWS_PER_W = L // SUBS_PER_B
CHUNK = 8
NCHUNK = ROWS_PER_W // CHUNK
NSLOT = 3
LANES = 16


def _sc_body(emb_hbm, mem_hbm, pos_hbm, out_hbm, buf, shared, pos_v,
             gsems, ssems):
    c = lax.axis_index("c")
    s = lax.axis_index("s")
    wid = c * NS + s
    b = wid // SUBS_PER_B
    sub = wid % SUBS_PER_B

    base_local = sub * ROWS_PER_W
    src_base = b * L + base_local
    out_base = b * NEW_L
    iota = lax.iota(jnp.int32, LANES)

    my_sh = s * CHUNK
    slots = [
        buf.at[pl.ds(0, CHUNK)],
        buf.at[pl.ds(CHUNK, CHUNK)],
        shared.at[pl.ds(my_sh, CHUNK)],
    ]

    def start_gather(i, k):
        r = pl.multiple_of(src_base + i * CHUNK, CHUNK)
        pltpu.async_copy(emb_hbm.at[pl.ds(r, CHUNK)], slots[k], gsems.at[k])

    def wait_gather(k):
        pltpu.make_async_copy(emb_hbm.at[pl.ds(0, CHUNK)], slots[k],
                              gsems.at[k]).wait()

    start_gather(0, 0)
    start_gather(1, 1)
    pltpu.sync_copy(pos_hbm, pos_v)
    pos_s = pos_v[pl.ds(b, LANES)][0]
    pos_vec = jnp.full((LANES,), pos_s, jnp.int32)

    def scatter_each(i, k, fn):
        row0 = base_local + i * CHUNK

        @pl.when(row0 < pos_s)
        def _():
            fn(slots[k], pl.multiple_of(out_base + row0, CHUNK), ssems.at[k])

        @pl.when(row0 + CHUNK > pos_s)
        def _():
            fn(slots[k], pl.multiple_of(out_base + row0 + N, CHUNK),
               ssems.at[k])

    def start_scatter(i, k):
        scatter_each(
            i, k, lambda sl, dst0, sem:
            pltpu.async_copy(sl, out_hbm.at[pl.ds(dst0, CHUNK)], sem))

    def wait_scatter(i, k):
        scatter_each(
            i, k, lambda sl, dst0, sem:
            pltpu.make_async_copy(sl, out_hbm.at[pl.ds(dst0, CHUNK)],
                                  sem).wait())

    for i in range(NCHUNK):
        wait_gather(i % NSLOT)
        start_scatter(i, i % NSLOT)
        if i >= 1:
            wait_scatter(i - 1, (i - 1) % NSLOT)
        if i + 2 < NCHUNK:
            start_gather(i + 2, (i + 2) % NSLOT)
    wait_scatter(NCHUNK - 1, (NCHUNK - 1) % NSLOT)

    @pl.when(sub == pos_s // ROWS_PER_W)
    def _():
        mrow0 = pl.multiple_of(b * N, N)
        stage = buf.at[pl.ds(0, N)]
        pltpu.sync_copy(mem_hbm.at[pl.ds(mrow0, N)], stage)
        dstm = out_base + pos_vec + iota
        pltpu.sync_copy(stage, out_hbm.at[dstm])


def _mask_body(am_ref, pos_ref, out_ref):
    j = lax.broadcasted_iota(jnp.int32, (B, NEW_L), 1)
    pos = pos_ref[...].reshape(B, 1)
    am = am_ref[...]
    zpad = jnp.zeros((B, N), jnp.float32)
    am_lo = jnp.concatenate([am, zpad], axis=1)
    am_hi = jnp.concatenate([zpad, am], axis=1)
    out_ref[...] = jnp.where(
        j < pos, am_lo, jnp.where(j >= pos + N, am_hi,
                                  jnp.ones((B, NEW_L), jnp.float32)))


@jax.jit
def kernel(inputs_embeds, memory, attention_mask, injection_positions):
    emb_flat = inputs_embeds.reshape(B * L, D)
    mem_flat = memory.reshape(B * N, D)
    am = attention_mask.astype(jnp.float32)
    pos32 = injection_positions.astype(jnp.int32)
    pos_pad = jnp.zeros((2 * LANES,), jnp.int32).at[:B].set(pos32)

    mesh = plsc.VectorSubcoreMesh(core_axis_name="c", subcore_axis_name="s",
                                  num_cores=NC, num_subcores=NS)
    run = pl.kernel(
        _sc_body,
        out_type=jax.ShapeDtypeStruct((B * NEW_L, D), jnp.float32),
        mesh=mesh,
        scratch_types=[
            pltpu.VMEM((2 * CHUNK, D), jnp.float32),
            pltpu.MemorySpace.VMEM_SHARED((NS * CHUNK, D), jnp.float32),
            pltpu.VMEM((2 * LANES,), jnp.int32),
            pltpu.SemaphoreType.DMA((NSLOT,)),
            pltpu.SemaphoreType.DMA((NSLOT,)),
        ],
    )
    out_flat = run(emb_flat, mem_flat, pos_pad)

    new_mask = pl.pallas_call(
        _mask_body,
        out_shape=jax.ShapeDtypeStruct((B, NEW_L), jnp.float32),
    )(am, pos32)

    return out_flat.reshape(B, NEW_L, D), new_mask


# speedup vs baseline: 3.2494x; 1.0118x over previous
import jax
import jax.numpy as jnp
from jax import lax
from jax.experimental import pallas as pl
from jax.experimental.pallas import tpu as pltpu
from jax.experimental.pallas import tpu_sc as plsc

B, L, D, N = 4, 2048, 4096, 16
NEW_L = L + N
NC, NS = 2, 16
NW = NC * NS
SUBS_PER_B = NW // B
ROWS_PER_W = L // SUBS_PER_B
CHUNK = 8
NCHUNK = ROWS_PER_W // CHUNK
NSLOT = 3
LANES = 16


def _sc_body(emb_hbm, mem_hbm, pos_hbm, out_hbm, buf, pos_v, gsems, ssems):
    c = lax.axis_index("c")
    s = lax.axis_index("s")
    wid = c * NS + s
    b = wid // SUBS_PER_B
    sub = wid % SUBS_PER_B

    base_local = sub * ROWS_PER_W
    src_base = b * L + base_local
    out_base = b * NEW_L
    iota = lax.iota(jnp.int32, LANES)

    slots = [buf.at[pl.ds(k * CHUNK, CHUNK)] for k in range(NSLOT)]

    def start_gather(i, k):
        r = pl.multiple_of(src_base + i * CHUNK, CHUNK)
        pltpu.async_copy(emb_hbm.at[pl.ds(r, CHUNK)], slots[k], gsems.at[k])

    def wait_gather(k):
        pltpu.make_async_copy(emb_hbm.at[pl.ds(0, CHUNK)], slots[k],
                              gsems.at[k]).wait()

    start_gather(0, 0)
    start_gather(1, 1)
    pltpu.sync_copy(pos_hbm, pos_v)
    pos_s = pos_v[pl.ds(b, LANES)][0]
    pos_vec = jnp.full((LANES,), pos_s, jnp.int32)

    def scatter_each(i, k, fn):
        row0 = base_local + i * CHUNK

        @pl.when(row0 < pos_s)
        def _():
            fn(slots[k], pl.multiple_of(out_base + row0, CHUNK), ssems.at[k])

        @pl.when(row0 + CHUNK > pos_s)
        def _():
            fn(slots[k], pl.multiple_of(out_base + row0 + N, CHUNK),
               ssems.at[k])

    def start_scatter(i, k):
        scatter_each(
            i, k, lambda sl, dst0, sem:
            pltpu.async_copy(sl, out_hbm.at[pl.ds(dst0, CHUNK)], sem))

    def wait_scatter(i, k):
        scatter_each(
            i, k, lambda sl, dst0, sem:
            pltpu.make_async_copy(sl, out_hbm.at[pl.ds(dst0, CHUNK)],
                                  sem).wait())

    for i in range(NCHUNK):
        wait_gather(i % NSLOT)
        start_scatter(i, i % NSLOT)
        if i >= 1:
            wait_scatter(i - 1, (i - 1) % NSLOT)
        if i + 2 < NCHUNK:
            start_gather(i + 2, (i + 2) % NSLOT)
    wait_scatter(NCHUNK - 1, (NCHUNK - 1) % NSLOT)

    @pl.when(sub == pos_s // ROWS_PER_W)
    def _():
        mrow0 = pl.multiple_of(b * N, N)
        stage = buf.at[pl.ds(0, N)]
        pltpu.sync_copy(mem_hbm.at[pl.ds(mrow0, N)], stage)
        dstm = out_base + pos_vec + iota
        pltpu.sync_copy(stage, out_hbm.at[dstm])


def _mask_body(am_ref, pos_ref, out_ref):
    j = lax.broadcasted_iota(jnp.int32, (B, NEW_L), 1)
    pos = pos_ref[...].reshape(B, 1)
    am = am_ref[...]
    zpad = jnp.zeros((B, N), jnp.float32)
    am_lo = jnp.concatenate([am, zpad], axis=1)
    am_hi = jnp.concatenate([zpad, am], axis=1)
    out_ref[...] = jnp.where(
        j < pos, am_lo, jnp.where(j >= pos + N, am_hi,
                                  jnp.ones((B, NEW_L), jnp.float32)))


@jax.jit
def kernel(inputs_embeds, memory, attention_mask, injection_positions):
    emb_flat = inputs_embeds.reshape(B * L, D)
    mem_flat = memory.reshape(B * N, D)
    am = attention_mask.astype(jnp.float32)
    pos32 = injection_positions.astype(jnp.int32)
    pos_pad = jnp.zeros((2 * LANES,), jnp.int32).at[:B].set(pos32)

    mesh = plsc.VectorSubcoreMesh(core_axis_name="c", subcore_axis_name="s",
                                  num_cores=NC, num_subcores=NS)
    run = pl.kernel(
        _sc_body,
        out_type=jax.ShapeDtypeStruct((B * NEW_L, D), jnp.float32),
        mesh=mesh,
        scratch_types=[
            pltpu.VMEM((NSLOT * CHUNK, D), jnp.float32),
            pltpu.VMEM((2 * LANES,), jnp.int32),
            pltpu.SemaphoreType.DMA((NSLOT,)),
            pltpu.SemaphoreType.DMA((NSLOT,)),
        ],
    )
    out_flat = run(emb_flat, mem_flat, pos_pad)

    new_mask = pl.pallas_call(
        _mask_body,
        out_shape=jax.ShapeDtypeStruct((B, NEW_L), jnp.float32),
    )(am, pos32)

    return out_flat.reshape(B, NEW_L, D), new_mask
